# Initial kernel scaffold; baseline (speedup 1.0000x reference)
import jax
import jax.numpy as jnp
from jax.experimental import pallas as pl


def kernel(center, outside, all_vocabs, emb_center, emb_outside):
    raise NotImplementedError("write your pallas kernel here")


# speedup vs baseline: 73.8392x; 73.8392x over previous
import functools
import jax
import jax.numpy as jnp
from jax import lax
from jax.experimental import pallas as pl
from jax.experimental.pallas import tpu as pltpu
from jax.experimental.pallas import tpu_sc as plsc

B = 1024
V = 1000
D = 64
DP = 128
VP = 1024
L = 16
NC, NS = 2, 16
NW = NC * NS
BPW = B // NW
NCHUNK = VP // L

_sc_mesh = plsc.VectorSubcoreMesh(core_axis_name="c", subcore_axis_name="s")


@functools.partial(
    pl.kernel,
    out_type=(jax.ShapeDtypeStruct((B, DP), jnp.float32),
              jax.ShapeDtypeStruct((B, DP), jnp.float32)),
    mesh=_sc_mesh,
    scratch_types=[
        pltpu.VMEM((BPW,), jnp.int32),
        pltpu.VMEM((BPW, DP), jnp.float32),
        pltpu.SemaphoreType.DMA,
    ],
)
def _sc_row_gather(emb_c_hbm, emb_o_hbm, cidx_hbm, oidx_hbm,
                   c_out, o_out, idx_v, rows_v, sem):
    wid = lax.axis_index("s") * NC + lax.axis_index("c")
    base = wid * BPW
    pltpu.sync_copy(cidx_hbm.at[pl.ds(base, BPW)], idx_v)
    pltpu.async_copy(emb_c_hbm.at[idx_v], rows_v, sem).wait()
    pltpu.sync_copy(rows_v, c_out.at[pl.ds(base, BPW)])
    pltpu.sync_copy(oidx_hbm.at[pl.ds(base, BPW)], idx_v)
    pltpu.async_copy(emb_o_hbm.at[idx_v], rows_v, sem).wait()
    pltpu.sync_copy(rows_v, o_out.at[pl.ds(base, BPW)])


def _tc_scores_body(c_ref, eo_ref, es_ref):
    s = lax.dot_general(c_ref[...], eo_ref[...], (((1,), (1,)), ((), ())),
                        preferred_element_type=jnp.float32)
    col = lax.broadcasted_iota(jnp.int32, (B, VP), 1)
    es_ref[...] = jnp.where(col < V, jnp.exp(s), 0.0)


_tc_scores = pl.pallas_call(
    _tc_scores_body,
    out_shape=jax.ShapeDtypeStruct((B, VP), jnp.float32),
)


@functools.partial(
    pl.kernel,
    out_type=jax.ShapeDtypeStruct((B * L,), jnp.float32),
    mesh=_sc_mesh,
    scratch_types=[
        pltpu.VMEM((BPW * VP,), jnp.float32),
        pltpu.VMEM((BPW * VP,), jnp.int32),
        pltpu.VMEM((BPW * L,), jnp.float32),
    ],
    compiler_params=pltpu.CompilerParams(needs_layout_passes=False),
)
def _sc_gather_sum(es_hbm, idx_hbm, out_hbm, es_v, idx_v, acc_v):
    wid = lax.axis_index("s") * NC + lax.axis_index("c")
    base = wid * BPW
    pltpu.sync_copy(es_hbm.at[pl.ds(base * VP, BPW * VP)], es_v)
    pltpu.sync_copy(idx_hbm.at[pl.ds(base * VP, BPW * VP)], idx_v)

    def row(r, carry):
        roff = r * VP
        acc = jnp.zeros((L,), jnp.float32)
        for k in range(NCHUNK):
            iv = idx_v[pl.ds(roff + k * L, L)]
            acc = acc + plsc.load_gather(es_v, [iv + roff])
        acc_v[pl.ds(r * L, L)] = acc
        return carry

    lax.fori_loop(0, BPW, row, 0)
    pltpu.sync_copy(acc_v, out_hbm.at[pl.ds(base * L, BPW * L)])


def _tc_loss_body(c_ref, o_ref, part_ref, out_ref):
    top_log = jnp.sum(c_ref[...] * o_ref[...], axis=1)
    lsum = jnp.sum(part_ref[...], axis=1)
    out_ref[...] = (-jnp.mean(top_log - jnp.log(lsum))).reshape(1, 1)


_tc_loss = pl.pallas_call(
    _tc_loss_body,
    out_shape=jax.ShapeDtypeStruct((1, 1), jnp.float32),
)


def kernel(center, outside, all_vocabs, emb_center, emb_outside):
    cidx = center.reshape(B).astype(jnp.int32)
    oidx = outside.reshape(B).astype(jnp.int32)
    av = jnp.pad(all_vocabs.astype(jnp.int32), ((0, 0), (0, VP - V)),
                 constant_values=VP - 1)
    eo_pad = jnp.pad(emb_outside, ((0, VP - V), (0, DP - D)))
    ec_pad = jnp.pad(emb_center, ((0, 0), (0, DP - D)))

    c_rows, o_rows = _sc_row_gather(ec_pad, eo_pad, cidx, oidx)
    es = _tc_scores(c_rows, eo_pad)
    part = _sc_gather_sum(es.reshape(B * VP), av.reshape(B * VP))
    loss = _tc_loss(c_rows, o_rows, part.reshape(B, L))
    return loss.reshape(())
